# Initial kernel scaffold; baseline (speedup 1.0000x reference)
import jax
import jax.numpy as jnp
from jax.experimental import pallas as pl


def kernel(inputs):
    raise NotImplementedError("write your pallas kernel here")


# speedup vs baseline: 22.8925x; 22.8925x over previous
import jax
import jax.numpy as jnp
from jax import lax
from jax.experimental import pallas as pl
from jax.experimental.pallas import tpu as pltpu
from jax.experimental.pallas import tpu_sc as plsc

KTOP = 8
B, S, C = 4, 4096, 1024
LANES = 16
BLK = 16
NBLK = S // BLK
CGROUPS = C // LANES
NUNITS = B * CGROUPS
NWORKERS = 32
UNITS_PER_W = NUNITS // NWORKERS


def _kmax_body(x_hbm, out_hbm, data_v, bmax_v, outb_v):
  wid = lax.axis_index("s") * 2 + lax.axis_index("c")
  iota = lax.iota(jnp.int32, LANES)
  neg = jnp.full((LANES,), -jnp.inf, jnp.float32)
  zeros_i = jnp.zeros((LANES,), jnp.int32)

  def unit_body(gi, _):
    g = wid * UNITS_PER_W + gi
    b = g // CGROUPS
    cg = g % CGROUPS

    pltpu.sync_copy(x_hbm.at[b, :, pl.ds(cg * LANES, LANES)], data_v)

    def p1(k, _):
      r0 = k * BLK
      m = data_v[r0, :]
      for i in range(1, BLK):
        m = jnp.maximum(m, data_v[r0 + i, :])
      bmax_v[k, :] = m
      return 0

    lax.fori_loop(0, NBLK, p1, 0)

    def p2(k, carry):
      ks = list(carry[:KTOP])
      idxs = list(carry[KTOP:])
      v = bmax_v[k, :]
      vi = jnp.broadcast_to(k, (LANES,)).astype(jnp.int32)
      for j in range(KTOP):
        m = v > ks[j]
        nk = jnp.where(m, v, ks[j])
        ni = jnp.where(m, vi, idxs[j])
        v = jnp.where(m, ks[j], v)
        vi = jnp.where(m, idxs[j], vi)
        ks[j] = nk
        idxs[j] = ni
      return tuple(ks) + tuple(idxs)

    carry0 = (neg,) * KTOP + (zeros_i,) * KTOP
    carry = lax.fori_loop(0, NBLK, p2, carry0)
    winners = carry[KTOP:]

    def p3(i, accs):
      accs = list(accs)
      for p in range(KTOP):
        row = winners[p] * BLK + i
        y = plsc.load_gather(data_v, [row, iota])
        for j in range(KTOP):
          hi = jnp.maximum(accs[j], y)
          y = jnp.minimum(accs[j], y)
          accs[j] = hi
      return tuple(accs)

    accs = lax.fori_loop(0, BLK, p3, (neg,) * KTOP)

    for j in range(KTOP):
      plsc.store_scatter(outb_v, [iota * KTOP + j], accs[j])
    pltpu.sync_copy(outb_v, out_hbm.at[b, pl.ds(cg * LANES * KTOP, LANES * KTOP)])
    return 0

  lax.fori_loop(0, UNITS_PER_W, unit_body, 0)


def kernel(inputs):
  mesh = plsc.VectorSubcoreMesh(core_axis_name="c", subcore_axis_name="s")
  return pl.kernel(
      _kmax_body,
      out_type=jax.ShapeDtypeStruct((B, C * KTOP), jnp.float32),
      mesh=mesh,
      compiler_params=pltpu.CompilerParams(
          use_tc_tiling_on_sc=False, needs_layout_passes=False),
      scratch_types=[
          pltpu.VMEM((S, LANES), jnp.float32),
          pltpu.VMEM((NBLK, LANES), jnp.float32),
          pltpu.VMEM((LANES * KTOP,), jnp.float32),
      ],
  )(inputs)


# speedup vs baseline: 25.9925x; 1.1354x over previous
import jax
import jax.numpy as jnp
from jax import lax
from jax.experimental import pallas as pl
from jax.experimental.pallas import tpu as pltpu
from jax.experimental.pallas import tpu_sc as plsc

KTOP = 8
B, S, C = 4, 4096, 1024
LANES = 16
BLK = 16
NBLK = S // BLK
CGROUPS = C // LANES
NUNITS = B * CGROUPS
NWORKERS = 32
UNITS_PER_W = NUNITS // NWORKERS


def _kmax_body(x_hbm, out_hbm, data_v, outb_v):
  wid = lax.axis_index("s") * 2 + lax.axis_index("c")
  iota = lax.iota(jnp.int32, LANES)
  neg = jnp.full((LANES,), -jnp.inf, jnp.float32)
  zeros_i = jnp.zeros((LANES,), jnp.int32)

  def unit_body(gi, _):
    g = wid * UNITS_PER_W + gi
    b = g // CGROUPS
    cg = g % CGROUPS

    pltpu.sync_copy(x_hbm.at[b, :, pl.ds(cg * LANES, LANES)], data_v)

    def p12(k, carry):
      ks = list(carry[:KTOP])
      idxs = list(carry[KTOP:])
      r0 = k * BLK
      v = data_v[r0, :]
      for i in range(1, BLK):
        v = jnp.maximum(v, data_v[r0 + i, :])
      vi = jnp.broadcast_to(k, (LANES,)).astype(jnp.int32)
      for j in range(KTOP):
        m = v > ks[j]
        nk = jnp.where(m, v, ks[j])
        ni = jnp.where(m, vi, idxs[j])
        v = jnp.where(m, ks[j], v)
        vi = jnp.where(m, idxs[j], vi)
        ks[j] = nk
        idxs[j] = ni
      return tuple(ks) + tuple(idxs)

    carry0 = (neg,) * KTOP + (zeros_i,) * KTOP
    carry = lax.fori_loop(0, NBLK, p12, carry0, unroll=2)
    winners = carry[KTOP:]

    def p3(i, accs):
      accs = list(accs)
      for p in range(KTOP):
        row = winners[p] * BLK + i
        y = plsc.load_gather(data_v, [row, iota])
        for j in range(KTOP):
          hi = jnp.maximum(accs[j], y)
          y = jnp.minimum(accs[j], y)
          accs[j] = hi
      return tuple(accs)

    accs = lax.fori_loop(0, BLK, p3, (neg,) * KTOP)

    for j in range(KTOP):
      plsc.store_scatter(outb_v, [iota * KTOP + j], accs[j])
    pltpu.sync_copy(outb_v, out_hbm.at[b, pl.ds(cg * LANES * KTOP, LANES * KTOP)])
    return 0

  lax.fori_loop(0, UNITS_PER_W, unit_body, 0)


def kernel(inputs):
  mesh = plsc.VectorSubcoreMesh(core_axis_name="c", subcore_axis_name="s")
  return pl.kernel(
      _kmax_body,
      out_type=jax.ShapeDtypeStruct((B, C * KTOP), jnp.float32),
      mesh=mesh,
      compiler_params=pltpu.CompilerParams(
          use_tc_tiling_on_sc=False, needs_layout_passes=False),
      scratch_types=[
          pltpu.VMEM((S, LANES), jnp.float32),
          pltpu.VMEM((LANES * KTOP,), jnp.float32),
      ],
  )(inputs)
